# baseline (speedup 1.0000x reference)
import functools

import jax
import jax.numpy as jnp
from jax import lax
from jax.experimental import pallas as pl
from jax.experimental.pallas import tpu as pltpu
from jax.experimental.pallas import tpu_sc as plsc

VOCAB = 1000000
B = 16384
D = 64
K = 20

NC = 2
NS = 16
NW = NC * NS
BW = B // NW
GB = 4
GROUPS = BW // GB
GROW = GB * K
NCHUNK = 64
NFC = BW // NCHUNK
CG = GROUPS // NFC

NB = -(-VOCAB // 128)
NBW = -(-NB // NW)
NPAD = NB * 64

_SC_PARAMS = pltpu.CompilerParams(
    use_tc_tiling_on_sc=True, needs_layout_passes=False)


def _mesh():
    return plsc.VectorSubcoreMesh(
        core_axis_name="c", subcore_axis_name="s", num_cores=NC,
        num_subcores=NS)


def _pack(ftT, ctT):
    @functools.partial(
        pl.kernel,
        out_type=(
            jax.ShapeDtypeStruct((NPAD, 128), jnp.float32),
            jax.ShapeDtypeStruct((NPAD, 128), jnp.float32),
        ),
        mesh=_mesh(),
        compiler_params=_SC_PARAMS,
        scratch_types=[
            pltpu.VMEM((2, D, 128), jnp.float32),
            pltpu.VMEM((2, D, 128), jnp.float32),
            pltpu.SemaphoreType.DMA,
            pltpu.SemaphoreType.DMA,
        ],
    )
    def k(ftT_hbm, ctT_hbm, fp_hbm, cp_hbm, in_v, out_v, sem_i, sem_o):
        wid = lax.axis_index("s") * NC + lax.axis_index("c")
        b0 = wid * NBW
        lane = lax.broadcasted_iota(jnp.int32, (16,), 0)

        vrem = VOCAB - (NB - 1) * 128

        for src, dst in ((ftT_hbm, fp_hbm), (ctT_hbm, cp_hbm)):

            def in_copies(blk, par, w):
                return [pltpu.make_async_copy(
                    src.at[pl.ds(tr * 8, 8), pl.ds(blk * 128, w)],
                    in_v.at[par, pl.ds(tr * 8, 8), pl.ds(0, w)], sem_i)
                    for tr in range(D // 8)]

            def fire_in(i, par, wait=False, ok=True):
                blk = b0 + i

                @pl.when((blk < NB - 1) & ok)
                def _():
                    for c in in_copies(blk, par, 128):
                        c.wait() if wait else c.start()

                @pl.when((blk == NB - 1) & ok)
                def _():
                    for c in in_copies(blk, par, vrem):
                        c.wait() if wait else c.start()

            fire_in(0, 0)

            def body(i, carry):
                par = lax.rem(i, 2)
                blk = b0 + i
                fire_in(i, par, wait=True)
                fire_in(i + 1, 1 - par, ok=i + 1 < NBW)

                @pl.when(blk < NB)
                def _():
                    @pl.when(i >= 2)
                    def _():
                        pltpu.make_async_copy(
                            out_v.at[par], dst.at[pl.ds(0, D)], sem_o).wait()

                    def trans_body(oi, tc):
                        for t in range(8):
                            d_vec = (t & 3) * 16 + lane
                            v_loc = 2 * oi + (1 if t >= 4 else 0)
                            v = plsc.load_gather(
                                in_v.at[par],
                                [d_vec, jnp.full((16,), v_loc, jnp.int32)])
                            out_v[par, oi, pl.ds(t * 16, 16)] = v
                        return tc

                    lax.fori_loop(0, D, trans_body, 0)
                    pltpu.make_async_copy(
                        out_v.at[par], dst.at[pl.ds(blk * 64, D)],
                        sem_o).start()
                return carry

            lax.fori_loop(0, NBW, body, 0)
            for _ in range(2):
                pltpu.make_async_copy(
                    out_v.at[0], dst.at[pl.ds(0, D)], sem_o).wait()

    return k(ftT, ctT)


def _sc_scores(fp, cp, fi, ci, nit):
    @functools.partial(
        pl.kernel,
        out_type=(
            jax.ShapeDtypeStruct((B // 8, 128), jnp.float32),
            jax.ShapeDtypeStruct((B // 8, 128), jnp.float32),
        ),
        mesh=_mesh(),
        compiler_params=_SC_PARAMS,
        scratch_types=[
            pltpu.VMEM((NFC, NCHUNK), jnp.int32),
            pltpu.VMEM((NFC, NCHUNK), jnp.int32),
            pltpu.VMEM((NFC, NCHUNK), jnp.int32),
            pltpu.VMEM((NFC, NCHUNK), jnp.int32),
            pltpu.VMEM((K, BW), jnp.int32),
            pltpu.VMEM((GROUPS, GROW), jnp.int32),
            pltpu.VMEM((GROUPS, GROW), jnp.int32),
            pltpu.VMEM((2, NCHUNK, 128), jnp.float32),
            pltpu.VMEM((2, NCHUNK, 128), jnp.float32),
            pltpu.VMEM((2, GROW, 128), jnp.float32),
            pltpu.VMEM((NCHUNK // 8, 128), jnp.float32),
            pltpu.VMEM((NCHUNK // 8, 128), jnp.float32),
            pltpu.SemaphoreType.DMA,
            pltpu.SemaphoreType.DMA,
        ],
    )
    def k(fp_hbm, cp_hbm, fi_hbm, ci_hbm, nit_hbm, posi_hbm, neg_hbm,
          fidx_v, foff_v, cidx_v, coff_v, nk_v, nidx_v, noff_v,
          fe_v, ce_v, nce_v, posi_v, neg_v, sem_fc, sem_n):
        wid = lax.axis_index("s") * NC + lax.axis_index("c")
        base = wid * BW
        lane = lax.broadcasted_iota(jnp.int32, (16,), 0)

        for j in range(NFC):
            pltpu.sync_copy(
                fi_hbm.at[pl.ds(base + j * NCHUNK, NCHUNK)], fidx_v.at[j])
            pltpu.sync_copy(
                ci_hbm.at[pl.ds(base + j * NCHUNK, NCHUNK)], cidx_v.at[j])
        for j in range(NFC):
            for t in range(NCHUNK // 16):
                sl = pl.ds(t * 16, 16)
                v = fidx_v[j, sl]
                fidx_v[j, sl] = v >> 1
                foff_v[j, sl] = (v & 1) << 6
                v = cidx_v[j, sl]
                cidx_v[j, sl] = v >> 1
                coff_v[j, sl] = (v & 1) << 6
        for kk in range(K):
            pltpu.sync_copy(nit_hbm.at[kk, pl.ds(base, BW)], nk_v.at[kk])

        def regroup_body(g, carry):
            for t in range(GROW // 16):
                r = lane + (t * 16)
                bb = ((r >= K).astype(jnp.int32)
                      + (r >= 2 * K).astype(jnp.int32)
                      + (r >= 3 * K).astype(jnp.int32))
                kk = r - K * bb
                v = plsc.load_gather(nk_v, [kk, bb + g * GB])
                sl = pl.ds(t * 16, 16)
                nidx_v[g, sl] = v >> 1
                noff_v[g, sl] = (v & 1) << 6
            return carry

        lax.fori_loop(0, GROUPS, regroup_body, 0)

        pltpu.make_async_copy(
            fp_hbm.at[fidx_v.at[0]], fe_v.at[0], sem_fc).start()
        pltpu.make_async_copy(
            cp_hbm.at[cidx_v.at[0]], ce_v.at[0], sem_fc).start()
        pltpu.make_async_copy(
            cp_hbm.at[nidx_v.at[0]], nce_v.at[0], sem_n).start()

        def chunk_body(cc, ccarry):
            cb = lax.rem(cc, 2)
            pltpu.make_async_copy(
                fp_hbm.at[fidx_v.at[cc]], fe_v.at[cb], sem_fc).wait()
            pltpu.make_async_copy(
                cp_hbm.at[cidx_v.at[cc]], ce_v.at[cb], sem_fc).wait()

            @pl.when(cc < NFC - 1)
            def _():
                pltpu.make_async_copy(
                    fp_hbm.at[fidx_v.at[cc + 1]],
                    fe_v.at[1 - cb], sem_fc).start()
                pltpu.make_async_copy(
                    cp_hbm.at[cidx_v.at[cc + 1]],
                    ce_v.at[1 - cb], sem_fc).start()

            def step_body(s, carry):
                foffs = foff_v[cc, pl.ds(s * 16, 16)]
                coffs = coff_v[cc, pl.ds(s * 16, 16)]
                for u in range(4):
                    g2 = s * 4 + u
                    g = cc * CG + g2
                    par = u & 1
                    pltpu.make_async_copy(
                        cp_hbm.at[nidx_v.at[g]], nce_v.at[par], sem_n).wait()

                    @pl.when(g < GROUPS - 1)
                    def _():
                        pltpu.make_async_copy(
                            cp_hbm.at[nidx_v.at[g + 1]],
                            nce_v.at[1 - par], sem_n).start()

                    noffs = [noff_v[g, pl.ds(t * 16, 16)]
                             for t in range(GROW // 16)]
                    for bb in range(GB):
                        bl = g2 * GB + bb
                        ln = u * GB + bb
                        fo = foffs[ln]
                        f = [fe_v[cb, bl, pl.ds(fo + j * 16, 16)]
                             for j in range(4)]
                        acc = [jnp.zeros((16,), jnp.float32)
                               for _ in range(4)]
                        for kk in range(K):
                            r = bb * K + kk
                            no = noffs[r // 16][r % 16]
                            for j in range(4):
                                acc[j] = (acc[j]
                                          + nce_v[par, r,
                                                  pl.ds(no + j * 16, 16)]
                                          * f[j])
                        row = bl >> 3
                        col = (bl & 7) * 16
                        neg_v[row, pl.ds(col, 16)] = (
                            acc[0] + acc[1] + acc[2] + acc[3])
                        co = coffs[ln]
                        c = [ce_v[cb, bl, pl.ds(co + j * 16, 16)]
                             for j in range(4)]
                        posi_v[row, pl.ds(col, 16)] = (
                            c[0] * f[0] + c[1] * f[1]
                            + c[2] * f[2] + c[3] * f[3])
                return carry

            lax.fori_loop(0, CG // 4, step_body, 0)

            orow = wid * (BW // 8) + cc * (NCHUNK // 8)
            pltpu.sync_copy(
                posi_v, posi_hbm.at[pl.ds(orow, NCHUNK // 8)])
            pltpu.sync_copy(
                neg_v, neg_hbm.at[pl.ds(orow, NCHUNK // 8)])
            return ccarry

        lax.fori_loop(0, NFC, chunk_body, 0)

    return k(fp, cp, fi, ci, nit)


def _tc_loss_body(p_ref, n_ref, o_ref):
    i = lax.broadcasted_iota(jnp.int32, (128, 8), 0)
    j = lax.broadcasted_iota(jnp.int32, (128, 8), 1)
    m = jnp.where(i // 16 == j, 1.0, 0.0).astype(jnp.float32)
    dn = (((1,), (0,)), ((), ()))
    ps = lax.dot_general(p_ref[...], m, dn, precision=lax.Precision.HIGHEST)
    ns = lax.dot_general(n_ref[...], m, dn, precision=lax.Precision.HIGHEST)
    ls_p = jnp.minimum(ps, 0.0) - jnp.log1p(jnp.exp(-jnp.abs(ps)))
    ls_n = jnp.minimum(ns, 0.0) - jnp.log1p(jnp.exp(-jnp.abs(ns)))
    o_ref[0, 0] = jnp.sum(jnp.square(1.0 - ls_p)) + jnp.sum(jnp.square(ls_n))


def _tc_loss(posi_part, neg_part):
    out = pl.pallas_call(
        _tc_loss_body,
        out_shape=jax.ShapeDtypeStruct((1, 1), jnp.float32),
        in_specs=[
            pl.BlockSpec(memory_space=pltpu.VMEM),
            pl.BlockSpec(memory_space=pltpu.VMEM),
        ],
        out_specs=pl.BlockSpec(memory_space=pltpu.SMEM),
    )(posi_part, neg_part)
    return out.reshape(())


def kernel(focus_table, context_table, focus_idx, context_idx,
           neg_context_idx):
    fi = focus_idx.astype(jnp.int32)
    ci = context_idx.astype(jnp.int32)
    nit = neg_context_idx.astype(jnp.int32).T
    fp, cp = _pack(focus_table.T, context_table.T)
    posi, neg = _sc_scores(fp, cp, fi, ci, nit)
    return _tc_loss(posi, neg)


# speedup vs baseline: 2.6407x; 2.6407x over previous
import functools

import jax
import jax.numpy as jnp
from jax import lax
from jax.experimental import pallas as pl
from jax.experimental.pallas import tpu as pltpu
from jax.experimental.pallas import tpu_sc as plsc

B = 16384
D = 64
K = 20

NC = 2
NS = 16
NW = NC * NS
BW = B // NW
GB = 4
GROUPS = BW // GB
GROW = GB * K
NCHUNK = 128
NFC = BW // NCHUNK


def _sc_scores(focus_table, context_table, fi2, ci2, ni2):
    mesh = plsc.VectorSubcoreMesh(
        core_axis_name="c", subcore_axis_name="s", num_cores=NC,
        num_subcores=NS)

    @functools.partial(
        pl.kernel,
        out_type=(
            jax.ShapeDtypeStruct((B, 16), jnp.float32),
            jax.ShapeDtypeStruct((B, 16), jnp.float32),
        ),
        mesh=mesh,
        compiler_params=pltpu.CompilerParams(use_tc_tiling_on_sc=False),
        scratch_types=[
            pltpu.VMEM((NFC, NCHUNK), jnp.int32),
            pltpu.VMEM((NFC, NCHUNK), jnp.int32),
            pltpu.VMEM((GROUPS, GROW), jnp.int32),
            pltpu.VMEM((BW, D), jnp.float32),
            pltpu.VMEM((BW, D), jnp.float32),
            pltpu.VMEM((2, GROW, D), jnp.float32),
            pltpu.VMEM((BW, 16), jnp.float32),
            pltpu.VMEM((BW, 16), jnp.float32),
            pltpu.SemaphoreType.DMA,
            pltpu.SemaphoreType.DMA,
        ],
    )
    def k(ft_hbm, ct_hbm, fi_hbm, ci_hbm, ni_hbm, posi_hbm, neg_hbm,
          fidx_v, cidx_v, nidx_v, fe_v, ce_v, nce_v, posi_v, neg_v,
          sem_fc, sem_n):
        wid = lax.axis_index("s") * NC + lax.axis_index("c")
        base = wid * BW

        pltpu.sync_copy(fi_hbm.at[pl.ds(wid * NFC, NFC)], fidx_v)
        pltpu.sync_copy(ci_hbm.at[pl.ds(wid * NFC, NFC)], cidx_v)
        pltpu.sync_copy(ni_hbm.at[pl.ds(wid * GROUPS, GROUPS)], nidx_v)

        for j in range(NFC):
            pltpu.make_async_copy(
                ft_hbm.at[fidx_v.at[j]],
                fe_v.at[pl.ds(j * NCHUNK, NCHUNK)], sem_fc).start()
        for j in range(NFC):
            pltpu.make_async_copy(
                ct_hbm.at[cidx_v.at[j]],
                ce_v.at[pl.ds(j * NCHUNK, NCHUNK)], sem_fc).start()
        pltpu.make_async_copy(
            ct_hbm.at[nidx_v.at[0]], nce_v.at[0], sem_n).start()
        for j in range(NFC):
            pltpu.make_async_copy(
                ft_hbm.at[fidx_v.at[j]],
                fe_v.at[pl.ds(j * NCHUNK, NCHUNK)], sem_fc).wait()
            pltpu.make_async_copy(
                ct_hbm.at[cidx_v.at[j]],
                ce_v.at[pl.ds(j * NCHUNK, NCHUNK)], sem_fc).wait()

        def group_body(g, carry):
            par = lax.rem(g, 2)
            pltpu.make_async_copy(
                ct_hbm.at[nidx_v.at[g]], nce_v.at[par], sem_n).wait()

            @pl.when(g < GROUPS - 1)
            def _():
                pltpu.make_async_copy(
                    ct_hbm.at[nidx_v.at[g + 1]],
                    nce_v.at[1 - par], sem_n).start()

            for bb in range(GB):
                b = g * GB + bb
                f = [fe_v[b, pl.ds(j * 16, 16)] for j in range(4)]
                acc = [jnp.zeros((16,), jnp.float32) for _ in range(4)]
                for kk in range(K):
                    r = bb * K + kk
                    for j in range(4):
                        acc[j] = acc[j] + nce_v[par, r, pl.ds(j * 16, 16)] * f[j]
                neg_v[b, :] = acc[0] + acc[1] + acc[2] + acc[3]
                c = [ce_v[b, pl.ds(j * 16, 16)] for j in range(4)]
                posi_v[b, :] = (
                    c[0] * f[0] + c[1] * f[1] + c[2] * f[2] + c[3] * f[3])
            return carry

        lax.fori_loop(0, GROUPS, group_body, 0)

        pltpu.sync_copy(posi_v, posi_hbm.at[pl.ds(base, BW)])
        pltpu.sync_copy(neg_v, neg_hbm.at[pl.ds(base, BW)])

    return k(focus_table, context_table, fi2, ci2, ni2)


def _tc_loss_body(p_ref, n_ref, o_ref):
    i = lax.broadcasted_iota(jnp.int32, (128, 8), 0)
    j = lax.broadcasted_iota(jnp.int32, (128, 8), 1)
    m = jnp.where(i // 16 == j, 1.0, 0.0).astype(jnp.float32)
    dn = (((1,), (0,)), ((), ()))
    ps = lax.dot_general(p_ref[...], m, dn, precision=lax.Precision.HIGHEST)
    ns = lax.dot_general(n_ref[...], m, dn, precision=lax.Precision.HIGHEST)
    ls_p = jnp.minimum(ps, 0.0) - jnp.log1p(jnp.exp(-jnp.abs(ps)))
    ls_n = jnp.minimum(ns, 0.0) - jnp.log1p(jnp.exp(-jnp.abs(ns)))
    o_ref[0, 0] = jnp.sum(jnp.square(1.0 - ls_p)) + jnp.sum(jnp.square(ls_n))


def _tc_loss(posi_part, neg_part):
    out = pl.pallas_call(
        _tc_loss_body,
        out_shape=jax.ShapeDtypeStruct((1, 1), jnp.float32),
        in_specs=[
            pl.BlockSpec(memory_space=pltpu.VMEM),
            pl.BlockSpec(memory_space=pltpu.VMEM),
        ],
        out_specs=pl.BlockSpec(memory_space=pltpu.SMEM),
    )(posi_part.reshape(B // 8, 128), neg_part.reshape(B // 8, 128))
    return out.reshape(())


def kernel(focus_table, context_table, focus_idx, context_idx,
           neg_context_idx):
    fi2 = focus_idx.astype(jnp.int32).reshape(B // NCHUNK, NCHUNK)
    ci2 = context_idx.astype(jnp.int32).reshape(B // NCHUNK, NCHUNK)
    ni2 = neg_context_idx.astype(jnp.int32).reshape(B * K // GROW, GROW)
    posi, neg = _sc_scores(focus_table, context_table, fi2, ci2, ni2)
    return _tc_loss(posi, neg)
